# baseline (speedup 1.0000x reference)
import jax
import jax.numpy as jnp
from jax import lax
from jax.experimental import pallas as pl
from jax.experimental.pallas import tpu as pltpu

_N = 1024
_HID = 64
_M = 2
_SEM_HID = 128
_BI = 256
_NB = _N // _BI
_SLOPE = 0.2

_SEMW = 0
_GATW = 128
_GMW0 = 192
_GMW1 = 256
_FTT = 320
_FTB = 384
_VEC = 448


def _row(col):
    return jnp.transpose(col)


def _fused_kernel(adj_hbm, feat_ref, aw_ref, wc_ref, p128_ref, out_ref,
                  abuf, h_s, el_s, elb_s, erowb_s, ecol_s,
                  rhs1_s, rhs2_s, acc1_s, acc2_s, sem):
    f32 = jnp.float32
    j = pl.program_id(0)

    def copy(blk):
        return pltpu.make_async_copy(
            adj_hbm.at[:, pl.ds(blk * _BI, _BI), :],
            abuf.at[blk % 2], sem.at[blk % 2])

    @pl.when(j == 0)
    def _():
        copy(0).start()

    @pl.when(j + 1 < _NB)
    def _():
        copy(j + 1).start()

    def wpar(g):
        base = (_GATW, _GMW0, _GMW1)[g]
        v = _VEC + (0, 4, 7)[g]
        W = wc_ref[:, base:base + _HID]
        al = wc_ref[:, v:v + 1]
        ar = wc_ref[:, v + 1:v + 2]
        b = wc_ref[:, v + 2:v + 3]
        return W, al, ar, b

    @pl.when(j == 0)
    def _():
        feat = feat_ref[...]
        for g in range(3):
            W, al, ar, _ = wpar(g)
            h = jnp.dot(feat, W, preferred_element_type=f32)
            el = jnp.dot(h, al, preferred_element_type=f32)
            elmax = jnp.max(el)
            u1 = jnp.exp(el - elmax)
            u2 = jnp.exp(_SLOPE * (el - elmax))
            h_s[g] = h
            el_s[g] = el
            elb_s[g] = el.astype(jnp.bfloat16)
            erow = lax.dot_general(_row(ar), h, (((1,), (1,)), ((), ())),
                                   preferred_element_type=f32)
            erowb_s[g] = erow.astype(jnp.bfloat16)
            ecol_s[g] = jnp.dot(h, ar, preferred_element_type=f32)
            rhs1_s[g] = jnp.concatenate([h * u1, u1],
                                        axis=1).astype(jnp.bfloat16)
            rhs2_s[g] = jnp.concatenate([h * u2, u2],
                                        axis=1).astype(jnp.bfloat16)

    copy(j).wait()

    bf = jnp.bfloat16
    a0b = abuf[j % 2, 0, :, :].astype(bf)
    a1b = abuf[j % 2, 1, :, :].astype(bf)
    w = jax.nn.softmax(aw_ref[...].reshape(1, _M))
    a0e = jnp.where(w[0, 0] != 0.0, a0b, bf(0.0))
    a1e = jnp.where(w[0, 1] != 0.0, a1b, bf(0.0))
    cnt_m = jnp.maximum(a0e, a1e)

    dn = (((0,), (0,)), ((), ()))
    for g, cnt in ((0, cnt_m), (1, a0b), (2, a1b)):
        el_blk = elb_s[g, pl.ds(j * _BI, _BI), :]
        x = el_blk + erowb_s[g]
        m1 = jnp.where(x >= bf(0.0), cnt, bf(0.0))
        m2 = cnt - m1
        rhs1 = rhs1_s[g, pl.ds(j * _BI, _BI), :]
        rhs2 = rhs2_s[g, pl.ds(j * _BI, _BI), :]
        r1 = lax.dot_general(m1, rhs1, dn, preferred_element_type=f32)
        r2 = lax.dot_general(m2, rhs2, dn, preferred_element_type=f32)

        @pl.when(j == 0)
        def _():
            acc1_s[g] = r1
            acc2_s[g] = r2

        @pl.when(j > 0)
        def _():
            acc1_s[g] += r1
            acc2_s[g] += r2

    @pl.when(j == _NB - 1)
    def _():
        zs = []
        for g in range(3):
            _, _, _, b = wpar(g)
            h = h_s[g]
            el = el_s[g]
            elmax = jnp.max(el)
            er_col = ecol_s[g]
            t = elmax + er_col
            c = jnp.where(t >= 0.0, t, _SLOPE * t)
            f1 = jnp.exp(t - c)
            f2 = jnp.exp(_SLOPE * t - c)
            xd = el + er_col
            ed = jnp.where(xd >= 0.0, xd, _SLOPE * xd)
            term = jnp.exp(ed - c)
            A1 = acc1_s[g]
            A2 = acc2_s[g]
            num = f1 * A1[:, :_HID] + f2 * A2[:, :_HID] + term * h
            den = (f1 * A1[:, _HID:_HID + 1] + f2 * A2[:, _HID:_HID + 1]
                   + term)
            zs.append(jnp.tanh(num / den + _row(b)))
        mg, m0, m1_ = zs

        sem_W1 = wc_ref[:, _SEMW:_SEMW + _SEM_HID]
        sem_b1 = _row(p128_ref[:, 0:1])
        sem_q = _row(p128_ref[:, 1:2])

        def wp(xv):
            tt = jnp.tanh(jnp.dot(xv, sem_W1, preferred_element_type=f32)
                          + sem_b1)
            return jnp.sum(tt * sem_q)

        s0 = wp(mg) / _N
        s1 = wp(m0) / _N
        s2 = wp(m1_) / _N
        smax = jnp.maximum(jnp.maximum(s0, s1), s2)
        e0 = jnp.exp(s0 - smax)
        e1 = jnp.exp(s1 - smax)
        e2 = jnp.exp(s2 - smax)
        tot = e0 + e1 + e2
        semantic = (e0 / tot) * mg + (e1 / tot) * m0 + (e2 / tot) * m1_

        ftb = _row(wc_ref[:, _VEC + 3:_VEC + 4])
        fa = (jnp.dot(mg, wc_ref[:, _FTT:_FTT + _HID],
                      preferred_element_type=f32)
              + jnp.dot(semantic, wc_ref[:, _FTB:_FTB + _HID],
                        preferred_element_type=f32)
              + ftb)
        out_ref[...] = jnp.tanh(fa)


def kernel(adj_list, feat, attention_weights, gat_W, gat_al, gat_ar, gat_b,
           gm_W, gm_al, gm_ar, gm_b, sem_W1, sem_b1, sem_q, ft_W, ft_b):
    vecs = jnp.stack([gat_al, gat_ar, gat_b, ft_b,
                      gm_al[0], gm_ar[0], gm_b[0],
                      gm_al[1], gm_ar[1], gm_b[1]], axis=1)
    wc = jnp.concatenate([sem_W1, gat_W, gm_W[0], gm_W[1],
                          ft_W[0:_HID], ft_W[_HID:2 * _HID], vecs],
                         axis=1)
    p128 = jnp.stack([sem_b1, sem_q], axis=1)

    full = lambda shape: pl.BlockSpec(shape, lambda j: (0,) * len(shape))
    out = pl.pallas_call(
        _fused_kernel,
        grid=(_NB,),
        in_specs=[
            pl.BlockSpec(memory_space=pl.ANY),
            full((_N, _HID)),
            full((_M,)),
            full((_HID, 458)),
            full((_SEM_HID, 2)),
        ],
        out_specs=pl.BlockSpec((_N, _HID), lambda j: (0, 0)),
        out_shape=jax.ShapeDtypeStruct((_N, _HID), jnp.float32),
        scratch_shapes=[
            pltpu.VMEM((2, _M, _BI, _N), jnp.int32),
            pltpu.VMEM((3, _N, _HID), jnp.float32),
            pltpu.VMEM((3, _N, 1), jnp.float32),
            pltpu.VMEM((3, _N, 1), jnp.bfloat16),
            pltpu.VMEM((3, 1, _N), jnp.bfloat16),
            pltpu.VMEM((3, _N, 1), jnp.float32),
            pltpu.VMEM((3, _N, _HID + 1), jnp.bfloat16),
            pltpu.VMEM((3, _N, _HID + 1), jnp.bfloat16),
            pltpu.VMEM((3, _N, _HID + 1), jnp.float32),
            pltpu.VMEM((3, _N, _HID + 1), jnp.float32),
            pltpu.SemaphoreType.DMA((2,)),
        ],
    )(adj_list, feat, attention_weights, wc, p128)
    return out


# speedup vs baseline: 1.3429x; 1.3429x over previous
import jax
import jax.numpy as jnp
from jax import lax
from jax.experimental import pallas as pl
from jax.experimental.pallas import tpu as pltpu

_N = 1024
_HID = 64
_M = 2
_SEM_HID = 128
_BI = 256
_NB = _N // _BI
_SLOPE = 0.2


def _fused_kernel(adj_hbm, feat_ref, aw_ref, gat_W_ref, gat_al_ref,
                  gat_ar_ref, gat_b_ref, gm_W_ref, gm_al_ref, gm_ar_ref,
                  gm_b_ref, sem_W1_ref, sem_b1_ref, sem_q_ref, ft_W_ref,
                  ft_b_ref, out_ref, abuf, h_s, el_s, elrow_s, erow_s,
                  rhs1_s, rhs2_s, acc1_s, acc2_s, sem):
    f32 = jnp.float32
    j = pl.program_id(0)

    def copy(blk):
        return pltpu.make_async_copy(
            adj_hbm.at[:, pl.ds(blk * _BI, _BI), :],
            abuf.at[blk % 2], sem.at[blk % 2])

    @pl.when(j == 0)
    def _():
        copy(0).start()

    @pl.when(j + 1 < _NB)
    def _():
        copy(j + 1).start()

    @pl.when(j == 0)
    def _():
        feat = feat_ref[...]
        params = ((gat_W_ref[...], gat_al_ref[...].reshape(1, _HID),
                   gat_ar_ref[...].reshape(1, _HID)),
                  (gm_W_ref[0], gm_al_ref[0:1, :], gm_ar_ref[0:1, :]),
                  (gm_W_ref[1], gm_al_ref[1:2, :], gm_ar_ref[1:2, :]))
        rowdot = (((1,), (1,)), ((), ()))
        for g, (W, al, ar) in enumerate(params):
            h = jnp.dot(feat, W, preferred_element_type=f32)
            elr = lax.dot_general(al, h, rowdot,
                                  preferred_element_type=f32)
            elmax = jnp.max(elr)
            u1r = jnp.exp(elr - elmax)
            u2r = jnp.exp(_SLOPE * (elr - elmax))
            u1 = jnp.transpose(u1r)
            u2 = jnp.transpose(u2r)
            h_s[g] = h
            el_s[g] = jnp.transpose(elr)
            elrow_s[g] = elr
            erow_s[g] = lax.dot_general(ar, h, rowdot,
                                        preferred_element_type=f32)
            rhs1_s[g, :, 0:_HID] = (h * u1).astype(jnp.bfloat16)
            rhs1_s[g, :, _HID:_HID + 1] = u1.astype(jnp.bfloat16)
            rhs2_s[g, :, 0:_HID] = (h * u2).astype(jnp.bfloat16)
            rhs2_s[g, :, _HID:_HID + 1] = u2.astype(jnp.bfloat16)

    copy(j).wait()

    a0f = abuf[j % 2, 0, :, :].astype(f32)
    a1f = abuf[j % 2, 1, :, :].astype(f32)
    w = jax.nn.softmax(aw_ref[...].reshape(1, _M))
    mm = a0f * w[0:1, 0:1] + a1f * w[0:1, 1:2]
    cnt_m = jnp.where(mm != 0.0, 1.0, 0.0)

    dn = (((0,), (0,)), ((), ()))
    for g, cnt in ((0, cnt_m), (1, a0f), (2, a1f)):
        el_blk = el_s[g, pl.ds(j * _BI, _BI), :]
        x = el_blk + erow_s[g]
        m1f = jnp.where(x >= 0.0, cnt, 0.0)
        m1 = m1f.astype(jnp.bfloat16)
        m2 = (cnt - m1f).astype(jnp.bfloat16)
        rhs1 = rhs1_s[g, pl.ds(j * _BI, _BI), :]
        rhs2 = rhs2_s[g, pl.ds(j * _BI, _BI), :]
        r1 = lax.dot_general(m1, rhs1, dn, preferred_element_type=f32)
        r2 = lax.dot_general(m2, rhs2, dn, preferred_element_type=f32)

        @pl.when(j == 0)
        def _():
            acc1_s[g] = r1
            acc2_s[g] = r2

        @pl.when(j > 0)
        def _():
            acc1_s[g] += r1
            acc2_s[g] += r2

    @pl.when(j == _NB - 1)
    def _():
        zs = []
        for g in range(3):
            h = h_s[g]
            elr = elrow_s[g]
            err = erow_s[g]
            elmax = jnp.max(elr)
            t = elmax + err
            c = jnp.where(t >= 0.0, t, _SLOPE * t)
            f1r = jnp.exp(t - c)
            f2r = jnp.exp(_SLOPE * t - c)
            xd = elr + err
            ed = jnp.where(xd >= 0.0, xd, _SLOPE * xd)
            termr = jnp.exp(ed - c)
            f1 = jnp.transpose(f1r)
            f2 = jnp.transpose(f2r)
            term = jnp.transpose(termr)
            A1 = acc1_s[g]
            A2 = acc2_s[g]
            num = f1 * A1[:, :_HID] + f2 * A2[:, :_HID] + term * h
            den = (f1 * A1[:, _HID:_HID + 1] + f2 * A2[:, _HID:_HID + 1]
                   + term)
            zs.append(num / den)
        mg = jnp.tanh(zs[0] + gat_b_ref[...].reshape(1, _HID))
        m0 = jnp.tanh(zs[1] + gm_b_ref[0:1, :])
        m1_ = jnp.tanh(zs[2] + gm_b_ref[1:2, :])

        sem_W1 = sem_W1_ref[...]
        sem_b1 = sem_b1_ref[...].reshape(1, _SEM_HID)
        sem_q = sem_q_ref[...].reshape(1, _SEM_HID)

        def wp(xv):
            tt = jnp.tanh(jnp.dot(xv, sem_W1, preferred_element_type=f32)
                          + sem_b1)
            return jnp.sum(tt * sem_q)

        s0 = wp(mg) / _N
        s1 = wp(m0) / _N
        s2 = wp(m1_) / _N
        smax = jnp.maximum(jnp.maximum(s0, s1), s2)
        e0 = jnp.exp(s0 - smax)
        e1 = jnp.exp(s1 - smax)
        e2 = jnp.exp(s2 - smax)
        tot = e0 + e1 + e2
        semantic = (e0 / tot) * mg + (e1 / tot) * m0 + (e2 / tot) * m1_

        ft_W = ft_W_ref[...]
        fa = (jnp.dot(mg, ft_W[0:_HID, :], preferred_element_type=f32)
              + jnp.dot(semantic, ft_W[_HID:2 * _HID, :],
                        preferred_element_type=f32)
              + ft_b_ref[...].reshape(1, _HID))
        out_ref[...] = jnp.tanh(fa)


def kernel(adj_list, feat, attention_weights, gat_W, gat_al, gat_ar, gat_b,
           gm_W, gm_al, gm_ar, gm_b, sem_W1, sem_b1, sem_q, ft_W, ft_b):
    full = lambda shape: pl.BlockSpec(shape, lambda j: (0,) * len(shape))
    out = pl.pallas_call(
        _fused_kernel,
        grid=(_NB,),
        in_specs=[
            pl.BlockSpec(memory_space=pl.ANY),
            full((_N, _HID)),
            full((_M,)),
            full((_HID, _HID)),
            full((_HID,)),
            full((_HID,)),
            full((_HID,)),
            full((_M, _HID, _HID)),
            full((_M, _HID)),
            full((_M, _HID)),
            full((_M, _HID)),
            full((_HID, _SEM_HID)),
            full((_SEM_HID,)),
            full((_SEM_HID,)),
            full((2 * _HID, _HID)),
            full((_HID,)),
        ],
        out_specs=pl.BlockSpec((_N, _HID), lambda j: (0, 0)),
        out_shape=jax.ShapeDtypeStruct((_N, _HID), jnp.float32),
        scratch_shapes=[
            pltpu.VMEM((2, _M, _BI, _N), jnp.int32),
            pltpu.VMEM((3, _N, _HID), jnp.float32),
            pltpu.VMEM((3, _N, 1), jnp.float32),
            pltpu.VMEM((3, 1, _N), jnp.float32),
            pltpu.VMEM((3, 1, _N), jnp.float32),
            pltpu.VMEM((3, _N, _HID + 1), jnp.bfloat16),
            pltpu.VMEM((3, _N, _HID + 1), jnp.bfloat16),
            pltpu.VMEM((3, _N, _HID + 1), jnp.float32),
            pltpu.VMEM((3, _N, _HID + 1), jnp.float32),
            pltpu.SemaphoreType.DMA((2,)),
        ],
    )(adj_list, feat, attention_weights, gat_W, gat_al, gat_ar, gat_b,
      gm_W, gm_al, gm_ar, gm_b, sem_W1, sem_b1, sem_q, ft_W, ft_b)
    return out


# speedup vs baseline: 1.6848x; 1.2546x over previous
import jax
import jax.numpy as jnp
from jax import lax
from jax.experimental import pallas as pl
from jax.experimental.pallas import tpu as pltpu

_N = 1024
_HID = 64
_M = 2
_SEM_HID = 128
_BI = 512
_NB = _N // _BI
_SLOPE = 0.2


def _fused_kernel(adj_hbm, feat_ref, aw_ref, gat_W_ref, gat_al_ref,
                  gat_ar_ref, gat_b_ref, gm_W_ref, gm_al_ref, gm_ar_ref,
                  gm_b_ref, sem_W1_ref, sem_b1_ref, sem_q_ref, ft_W_ref,
                  ft_b_ref, out_ref, abuf, h_s, el_s, erow_s, ecol_s,
                  rhs1_s, rhs2_s, acc1_s, acc2_s, sem):
    f32 = jnp.float32
    j = pl.program_id(0)

    def copy(blk):
        return pltpu.make_async_copy(
            adj_hbm.at[:, pl.ds(blk * _BI, _BI), :],
            abuf.at[blk % 2], sem.at[blk % 2])

    @pl.when(j == 0)
    def _():
        copy(0).start()

    @pl.when(j + 1 < _NB)
    def _():
        copy(j + 1).start()

    @pl.when(j == 0)
    def _():
        feat = feat_ref[...]
        params = ((gat_W_ref[...], gat_al_ref[...].reshape(1, _HID),
                   gat_ar_ref[...].reshape(1, _HID)),
                  (gm_W_ref[0], gm_al_ref[0:1, :], gm_ar_ref[0:1, :]),
                  (gm_W_ref[1], gm_al_ref[1:2, :], gm_ar_ref[1:2, :]))
        for g, (W, al, ar) in enumerate(params):
            h = jnp.dot(feat, W, preferred_element_type=f32)
            el = jnp.sum(h * al, axis=1, keepdims=True)
            elmax = jnp.max(el)
            u1 = jnp.exp(el - elmax)
            u2 = jnp.exp(_SLOPE * (el - elmax))
            h_s[g] = h
            el_s[g] = el
            erow_s[g] = lax.dot_general(ar, h, (((1,), (1,)), ((), ())),
                                        preferred_element_type=f32)
            ecol_s[g] = jnp.sum(h * ar, axis=1, keepdims=True)
            rhs1_s[g] = jnp.concatenate([h * u1, u1],
                                        axis=1).astype(jnp.bfloat16)
            rhs2_s[g] = jnp.concatenate([h * u2, u2],
                                        axis=1).astype(jnp.bfloat16)

    copy(j).wait()

    a0f = abuf[j % 2, 0, :, :].astype(f32)
    a1f = abuf[j % 2, 1, :, :].astype(f32)
    w = jax.nn.softmax(aw_ref[...].reshape(1, _M))
    mm = a0f * w[0:1, 0:1] + a1f * w[0:1, 1:2]
    cnt_m = jnp.where(mm != 0.0, 1.0, 0.0)

    dn = (((0,), (0,)), ((), ()))
    for g, cnt in ((0, cnt_m), (1, a0f), (2, a1f)):
        el_blk = el_s[g, pl.ds(j * _BI, _BI), :]
        x = el_blk + erow_s[g]
        m1f = jnp.where(x >= 0.0, cnt, 0.0)
        m1 = m1f.astype(jnp.bfloat16)
        m2 = (cnt - m1f).astype(jnp.bfloat16)
        rhs1 = rhs1_s[g, pl.ds(j * _BI, _BI), :]
        rhs2 = rhs2_s[g, pl.ds(j * _BI, _BI), :]
        r1 = lax.dot_general(m1, rhs1, dn, preferred_element_type=f32)
        r2 = lax.dot_general(m2, rhs2, dn, preferred_element_type=f32)

        @pl.when(j == 0)
        def _():
            acc1_s[g] = r1
            acc2_s[g] = r2

        @pl.when(j > 0)
        def _():
            acc1_s[g] += r1
            acc2_s[g] += r2

    @pl.when(j == _NB - 1)
    def _():
        zs = []
        for g in range(3):
            h = h_s[g]
            el = el_s[g]
            elmax = jnp.max(el)
            er_col = ecol_s[g]
            t = elmax + er_col
            c = jnp.where(t >= 0.0, t, _SLOPE * t)
            f1 = jnp.exp(t - c)
            f2 = jnp.exp(_SLOPE * t - c)
            xd = el + er_col
            ed = jnp.where(xd >= 0.0, xd, _SLOPE * xd)
            term = jnp.exp(ed - c)
            A1 = acc1_s[g]
            A2 = acc2_s[g]
            num = f1 * A1[:, :_HID] + f2 * A2[:, :_HID] + term * h
            den = (f1 * A1[:, _HID:_HID + 1] + f2 * A2[:, _HID:_HID + 1]
                   + term)
            zs.append(num / den)
        mg = jnp.tanh(zs[0] + gat_b_ref[...].reshape(1, _HID))
        m0 = jnp.tanh(zs[1] + gm_b_ref[0:1, :])
        m1_ = jnp.tanh(zs[2] + gm_b_ref[1:2, :])

        sem_W1 = sem_W1_ref[...]
        sem_b1 = sem_b1_ref[...].reshape(1, _SEM_HID)
        sem_q = sem_q_ref[...].reshape(1, _SEM_HID)

        def wp(xv):
            tt = jnp.tanh(jnp.dot(xv, sem_W1, preferred_element_type=f32)
                          + sem_b1)
            return jnp.sum(tt * sem_q)

        s0 = wp(mg) / _N
        s1 = wp(m0) / _N
        s2 = wp(m1_) / _N
        smax = jnp.maximum(jnp.maximum(s0, s1), s2)
        e0 = jnp.exp(s0 - smax)
        e1 = jnp.exp(s1 - smax)
        e2 = jnp.exp(s2 - smax)
        tot = e0 + e1 + e2
        semantic = (e0 / tot) * mg + (e1 / tot) * m0 + (e2 / tot) * m1_

        ft_W = ft_W_ref[...]
        fa = (jnp.dot(mg, ft_W[0:_HID, :], preferred_element_type=f32)
              + jnp.dot(semantic, ft_W[_HID:2 * _HID, :],
                        preferred_element_type=f32)
              + ft_b_ref[...].reshape(1, _HID))
        out_ref[...] = jnp.tanh(fa)


def kernel(adj_list, feat, attention_weights, gat_W, gat_al, gat_ar, gat_b,
           gm_W, gm_al, gm_ar, gm_b, sem_W1, sem_b1, sem_q, ft_W, ft_b):
    full = lambda shape: pl.BlockSpec(shape, lambda j: (0,) * len(shape))
    out = pl.pallas_call(
        _fused_kernel,
        grid=(_NB,),
        in_specs=[
            pl.BlockSpec(memory_space=pl.ANY),
            full((_N, _HID)),
            full((_M,)),
            full((_HID, _HID)),
            full((_HID,)),
            full((_HID,)),
            full((_HID,)),
            full((_M, _HID, _HID)),
            full((_M, _HID)),
            full((_M, _HID)),
            full((_M, _HID)),
            full((_HID, _SEM_HID)),
            full((_SEM_HID,)),
            full((_SEM_HID,)),
            full((2 * _HID, _HID)),
            full((_HID,)),
        ],
        out_specs=pl.BlockSpec((_N, _HID), lambda j: (0, 0)),
        out_shape=jax.ShapeDtypeStruct((_N, _HID), jnp.float32),
        scratch_shapes=[
            pltpu.VMEM((2, _M, _BI, _N), jnp.int32),
            pltpu.VMEM((3, _N, _HID), jnp.float32),
            pltpu.VMEM((3, _N, 1), jnp.float32),
            pltpu.VMEM((3, 1, _N), jnp.float32),
            pltpu.VMEM((3, _N, 1), jnp.float32),
            pltpu.VMEM((3, _N, _HID + 1), jnp.bfloat16),
            pltpu.VMEM((3, _N, _HID + 1), jnp.bfloat16),
            pltpu.VMEM((3, _N, _HID + 1), jnp.float32),
            pltpu.VMEM((3, _N, _HID + 1), jnp.float32),
            pltpu.SemaphoreType.DMA((2,)),
        ],
    )(adj_list, feat, attention_weights, gat_W, gat_al, gat_ar, gat_b,
      gm_W, gm_al, gm_ar, gm_b, sem_W1, sem_b1, sem_q, ft_W, ft_b)
    return out


# speedup vs baseline: 1.7999x; 1.0683x over previous
import jax
import jax.numpy as jnp
from jax import lax
from jax.experimental import pallas as pl
from jax.experimental.pallas import tpu as pltpu

_N = 1024
_HID = 64
_M = 2
_SEM_HID = 128
_BI = 1024
_NB = _N // _BI
_SLOPE = 0.2


def _fused_kernel(adj_hbm, feat_ref, aw_ref, gat_W_ref, gat_al_ref,
                  gat_ar_ref, gat_b_ref, gm_W_ref, gm_al_ref, gm_ar_ref,
                  gm_b_ref, sem_W1_ref, sem_b1_ref, sem_q_ref, ft_W_ref,
                  ft_b_ref, out_ref, abuf, h_s, el_s, erow_s, ecol_s,
                  rhs1_s, rhs2_s, acc1_s, acc2_s, sem):
    f32 = jnp.float32
    j = pl.program_id(0)

    def copy(blk):
        return pltpu.make_async_copy(
            adj_hbm.at[:, pl.ds(blk * _BI, _BI), :],
            abuf.at[blk % 2], sem.at[blk % 2])

    @pl.when(j == 0)
    def _():
        copy(0).start()

    @pl.when(j + 1 < _NB)
    def _():
        copy(j + 1).start()

    @pl.when(j == 0)
    def _():
        feat = feat_ref[...]
        params = ((gat_W_ref[...], gat_al_ref[...].reshape(1, _HID),
                   gat_ar_ref[...].reshape(1, _HID)),
                  (gm_W_ref[0], gm_al_ref[0:1, :], gm_ar_ref[0:1, :]),
                  (gm_W_ref[1], gm_al_ref[1:2, :], gm_ar_ref[1:2, :]))
        for g, (W, al, ar) in enumerate(params):
            h = jnp.dot(feat, W, preferred_element_type=f32)
            el = jnp.sum(h * al, axis=1, keepdims=True)
            elmax = jnp.max(el)
            u1 = jnp.exp(el - elmax)
            u2 = jnp.exp(_SLOPE * (el - elmax))
            h_s[g] = h
            el_s[g] = el
            erow_s[g] = lax.dot_general(ar, h, (((1,), (1,)), ((), ())),
                                        preferred_element_type=f32)
            ecol_s[g] = jnp.sum(h * ar, axis=1, keepdims=True)
            rhs1_s[g] = jnp.concatenate([h * u1, u1],
                                        axis=1).astype(jnp.bfloat16)
            rhs2_s[g] = jnp.concatenate([h * u2, u2],
                                        axis=1).astype(jnp.bfloat16)

    copy(j).wait()

    a0f = abuf[j % 2, 0, :, :].astype(f32)
    a1f = abuf[j % 2, 1, :, :].astype(f32)
    w = jax.nn.softmax(aw_ref[...].reshape(1, _M))
    mm = a0f * w[0:1, 0:1] + a1f * w[0:1, 1:2]
    cnt_m = jnp.where(mm != 0.0, 1.0, 0.0)

    dn = (((0,), (0,)), ((), ()))
    for g, cnt in ((0, cnt_m), (1, a0f), (2, a1f)):
        el_blk = el_s[g, pl.ds(j * _BI, _BI), :]
        x = el_blk + erow_s[g]
        m1f = jnp.where(x >= 0.0, cnt, 0.0)
        m1 = m1f.astype(jnp.bfloat16)
        m2 = (cnt - m1f).astype(jnp.bfloat16)
        rhs1 = rhs1_s[g, pl.ds(j * _BI, _BI), :]
        rhs2 = rhs2_s[g, pl.ds(j * _BI, _BI), :]
        r1 = lax.dot_general(m1, rhs1, dn, preferred_element_type=f32)
        r2 = lax.dot_general(m2, rhs2, dn, preferred_element_type=f32)

        @pl.when(j == 0)
        def _():
            acc1_s[g] = r1
            acc2_s[g] = r2

        @pl.when(j > 0)
        def _():
            acc1_s[g] += r1
            acc2_s[g] += r2

    @pl.when(j == _NB - 1)
    def _():
        zs = []
        for g in range(3):
            h = h_s[g]
            el = el_s[g]
            elmax = jnp.max(el)
            er_col = ecol_s[g]
            t = elmax + er_col
            c = jnp.where(t >= 0.0, t, _SLOPE * t)
            f1 = jnp.exp(t - c)
            f2 = jnp.exp(_SLOPE * t - c)
            xd = el + er_col
            ed = jnp.where(xd >= 0.0, xd, _SLOPE * xd)
            term = jnp.exp(ed - c)
            A1 = acc1_s[g]
            A2 = acc2_s[g]
            num = f1 * A1[:, :_HID] + f2 * A2[:, :_HID] + term * h
            den = (f1 * A1[:, _HID:_HID + 1] + f2 * A2[:, _HID:_HID + 1]
                   + term)
            zs.append(num / den)
        mg = jnp.tanh(zs[0] + gat_b_ref[...].reshape(1, _HID))
        m0 = jnp.tanh(zs[1] + gm_b_ref[0:1, :])
        m1_ = jnp.tanh(zs[2] + gm_b_ref[1:2, :])

        sem_W1 = sem_W1_ref[...]
        sem_b1 = sem_b1_ref[...].reshape(1, _SEM_HID)
        sem_q = sem_q_ref[...].reshape(1, _SEM_HID)

        def wp(xv):
            tt = jnp.tanh(jnp.dot(xv, sem_W1, preferred_element_type=f32)
                          + sem_b1)
            return jnp.sum(tt * sem_q)

        s0 = wp(mg) / _N
        s1 = wp(m0) / _N
        s2 = wp(m1_) / _N
        smax = jnp.maximum(jnp.maximum(s0, s1), s2)
        e0 = jnp.exp(s0 - smax)
        e1 = jnp.exp(s1 - smax)
        e2 = jnp.exp(s2 - smax)
        tot = e0 + e1 + e2
        semantic = (e0 / tot) * mg + (e1 / tot) * m0 + (e2 / tot) * m1_

        ft_W = ft_W_ref[...]
        fa = (jnp.dot(mg, ft_W[0:_HID, :], preferred_element_type=f32)
              + jnp.dot(semantic, ft_W[_HID:2 * _HID, :],
                        preferred_element_type=f32)
              + ft_b_ref[...].reshape(1, _HID))
        out_ref[...] = jnp.tanh(fa)


def kernel(adj_list, feat, attention_weights, gat_W, gat_al, gat_ar, gat_b,
           gm_W, gm_al, gm_ar, gm_b, sem_W1, sem_b1, sem_q, ft_W, ft_b):
    full = lambda shape: pl.BlockSpec(shape, lambda j: (0,) * len(shape))
    out = pl.pallas_call(
        _fused_kernel,
        grid=(_NB,),
        in_specs=[
            pl.BlockSpec(memory_space=pl.ANY),
            full((_N, _HID)),
            full((_M,)),
            full((_HID, _HID)),
            full((_HID,)),
            full((_HID,)),
            full((_HID,)),
            full((_M, _HID, _HID)),
            full((_M, _HID)),
            full((_M, _HID)),
            full((_M, _HID)),
            full((_HID, _SEM_HID)),
            full((_SEM_HID,)),
            full((_SEM_HID,)),
            full((2 * _HID, _HID)),
            full((_HID,)),
        ],
        out_specs=pl.BlockSpec((_N, _HID), lambda j: (0, 0)),
        out_shape=jax.ShapeDtypeStruct((_N, _HID), jnp.float32),
        scratch_shapes=[
            pltpu.VMEM((2, _M, _BI, _N), jnp.int32),
            pltpu.VMEM((3, _N, _HID), jnp.float32),
            pltpu.VMEM((3, _N, 1), jnp.float32),
            pltpu.VMEM((3, 1, _N), jnp.float32),
            pltpu.VMEM((3, _N, 1), jnp.float32),
            pltpu.VMEM((3, _N, _HID + 1), jnp.bfloat16),
            pltpu.VMEM((3, _N, _HID + 1), jnp.bfloat16),
            pltpu.VMEM((3, _N, _HID + 1), jnp.float32),
            pltpu.VMEM((3, _N, _HID + 1), jnp.float32),
            pltpu.SemaphoreType.DMA((2,)),
        ],
    )(adj_list, feat, attention_weights, gat_W, gat_al, gat_ar, gat_b,
      gm_W, gm_al, gm_ar, gm_b, sem_W1, sem_b1, sem_q, ft_W, ft_b)
    return out


# speedup vs baseline: 1.9443x; 1.0802x over previous
import jax
import jax.numpy as jnp
from jax import lax
from jax.experimental import pallas as pl
from jax.experimental.pallas import tpu as pltpu

_N = 1024
_HID = 64
_M = 2
_SEM_HID = 128
_CH = 256
_NC = _N // _CH
_SLOPE = 0.2


def _fused_kernel(adj_hbm, feat_ref, aw_ref, gat_W_ref, gat_al_ref,
                  gat_ar_ref, gat_b_ref, gm_W_ref, gm_al_ref, gm_ar_ref,
                  gm_b_ref, sem_W1_ref, sem_b1_ref, sem_q_ref, ft_W_ref,
                  ft_b_ref, out_ref, abuf, sem):
    f32 = jnp.float32
    bf = jnp.bfloat16

    def copy(k):
        return pltpu.make_async_copy(
            adj_hbm.at[:, pl.ds(k * _CH, _CH), :], abuf.at[k], sem.at[k])

    for k in range(_NC):
        copy(k).start()

    feat = feat_ref[...]
    params = ((gat_W_ref[...], gat_al_ref[...].reshape(1, _HID),
               gat_ar_ref[...].reshape(1, _HID)),
              (gm_W_ref[0], gm_al_ref[0:1, :], gm_ar_ref[0:1, :]),
              (gm_W_ref[1], gm_al_ref[1:2, :], gm_ar_ref[1:2, :]))
    H, EL, ELM, ER, EC, R1, R2 = [], [], [], [], [], [], []
    for W, al, ar in params:
        h = jnp.dot(feat, W, preferred_element_type=f32)
        el = jnp.sum(h * al, axis=1, keepdims=True)
        elmax = jnp.max(el)
        u1 = jnp.exp(el - elmax)
        u2 = jnp.exp(_SLOPE * (el - elmax))
        H.append(h)
        EL.append(el)
        ELM.append(elmax)
        ER.append(lax.dot_general(ar, h, (((1,), (1,)), ((), ())),
                                  preferred_element_type=f32))
        EC.append(jnp.sum(h * ar, axis=1, keepdims=True))
        R1.append(jnp.concatenate([h * u1, u1], axis=1).astype(bf))
        R2.append(jnp.concatenate([h * u2, u2], axis=1).astype(bf))

    w = jax.nn.softmax(aw_ref[...].reshape(1, _M))
    dn = (((0,), (0,)), ((), ()))
    A1 = [None] * 3
    A2 = [None] * 3
    for k in range(_NC):
        copy(k).wait()
        a0f = abuf[k, 0].astype(f32)
        a1f = abuf[k, 1].astype(f32)
        mm = a0f * w[0:1, 0:1] + a1f * w[0:1, 1:2]
        cnt_m = jnp.where(mm != 0.0, 1.0, 0.0)
        lo, hi = k * _CH, (k + 1) * _CH
        for g, cnt in ((0, cnt_m), (1, a0f), (2, a1f)):
            x = EL[g][lo:hi, :] + ER[g]
            m1f = jnp.where(x >= 0.0, cnt, 0.0)
            m1 = m1f.astype(bf)
            m2 = (cnt - m1f).astype(bf)
            r1 = lax.dot_general(m1, R1[g][lo:hi, :], dn,
                                 preferred_element_type=f32)
            r2 = lax.dot_general(m2, R2[g][lo:hi, :], dn,
                                 preferred_element_type=f32)
            A1[g] = r1 if k == 0 else A1[g] + r1
            A2[g] = r2 if k == 0 else A2[g] + r2

    zs = []
    for g in range(3):
        t = ELM[g] + EC[g]
        c = jnp.where(t >= 0.0, t, _SLOPE * t)
        f1 = jnp.exp(t - c)
        f2 = jnp.exp(_SLOPE * t - c)
        xd = EL[g] + EC[g]
        ed = jnp.where(xd >= 0.0, xd, _SLOPE * xd)
        term = jnp.exp(ed - c)
        num = f1 * A1[g][:, :_HID] + f2 * A2[g][:, :_HID] + term * H[g]
        den = (f1 * A1[g][:, _HID:_HID + 1] + f2 * A2[g][:, _HID:_HID + 1]
               + term)
        zs.append(num / den)
    mg = jnp.tanh(zs[0] + gat_b_ref[...].reshape(1, _HID))
    m0 = jnp.tanh(zs[1] + gm_b_ref[0:1, :])
    m1_ = jnp.tanh(zs[2] + gm_b_ref[1:2, :])

    sem_W1 = sem_W1_ref[...]
    sem_b1 = sem_b1_ref[...].reshape(1, _SEM_HID)
    sem_q = sem_q_ref[...].reshape(1, _SEM_HID)

    def wp(xv):
        tt = jnp.tanh(jnp.dot(xv, sem_W1, preferred_element_type=f32)
                      + sem_b1)
        return jnp.sum(tt * sem_q)

    s0 = wp(mg) / _N
    s1 = wp(m0) / _N
    s2 = wp(m1_) / _N
    smax = jnp.maximum(jnp.maximum(s0, s1), s2)
    e0 = jnp.exp(s0 - smax)
    e1 = jnp.exp(s1 - smax)
    e2 = jnp.exp(s2 - smax)
    tot = e0 + e1 + e2
    semantic = (e0 / tot) * mg + (e1 / tot) * m0 + (e2 / tot) * m1_

    ft_W = ft_W_ref[...]
    fa = (jnp.dot(mg, ft_W[0:_HID, :], preferred_element_type=f32)
          + jnp.dot(semantic, ft_W[_HID:2 * _HID, :],
                    preferred_element_type=f32)
          + ft_b_ref[...].reshape(1, _HID))
    out_ref[...] = jnp.tanh(fa)


def kernel(adj_list, feat, attention_weights, gat_W, gat_al, gat_ar, gat_b,
           gm_W, gm_al, gm_ar, gm_b, sem_W1, sem_b1, sem_q, ft_W, ft_b):
    vmem = lambda: pl.BlockSpec(memory_space=pltpu.MemorySpace.VMEM)
    out = pl.pallas_call(
        _fused_kernel,
        in_specs=[pl.BlockSpec(memory_space=pl.ANY)] + [vmem()] * 15,
        out_specs=vmem(),
        out_shape=jax.ShapeDtypeStruct((_N, _HID), jnp.float32),
        scratch_shapes=[
            pltpu.VMEM((_NC, _M, _CH, _N), jnp.int32),
            pltpu.SemaphoreType.DMA((_NC,)),
        ],
    )(adj_list, feat, attention_weights, gat_W, gat_al, gat_ar, gat_b,
      gm_W, gm_al, gm_ar, gm_b, sem_W1, sem_b1, sem_q, ft_W, ft_b)
    return out
